# baseline (speedup 1.0000x reference)
import functools

import jax
import jax.numpy as jnp
from jax import lax
from jax.experimental import pallas as pl
from jax.experimental.pallas import tpu as pltpu
from jax.experimental.pallas import tpu_sc as plsc

_NW = 32
_BW = 2
_NBUF = 4
_NCHUNK = 4
_PAD = 56
_BB = 64


def _make_gather(dtype, cb, hist, dim):
    mesh = plsc.VectorSubcoreMesh(
        core_axis_name="core", subcore_axis_name="subcore"
    )
    del hist
    bpw = cb // _NW
    nwin = bpw // _BW
    rows = _BW * _PAD

    @functools.partial(
        pl.kernel,
        mesh=mesh,
        out_type=jax.ShapeDtypeStruct((cb, _PAD, dim), dtype),
        scratch_types=[
            pltpu.VMEM((nwin, rows), jnp.int32),
        ]
        + [pltpu.VMEM((rows, dim), dtype) for _ in range(_NBUF)]
        + [pltpu.SemaphoreType.DMA for _ in range(2 * _NBUF)],
    )
    def gather_kernel(w_hbm, x_hbm, o_hbm, idx_v, *rest):
        bufs = rest[:_NBUF]
        gsems = rest[_NBUF:2 * _NBUF]
        osems = rest[2 * _NBUF:]

        wid = lax.axis_index("subcore") * 2 + lax.axis_index("core")
        base = wid * bpw
        pltpu.sync_copy(x_hbm.at[wid], idx_v)

        def start_gather(j, b):
            pltpu.async_copy(w_hbm.at[idx_v.at[j]], bufs[b], gsems[b])

        def wait_gather(b):
            pltpu.make_async_copy(
                w_hbm.at[idx_v.at[0]], bufs[b], gsems[b]
            ).wait()

        def start_out(j, b):
            b0 = base + j * _BW
            for k in range(_BW):
                pltpu.async_copy(
                    bufs[b].at[pl.ds(k * _PAD, _PAD)],
                    o_hbm.at[b0 + k],
                    osems[b],
                )

        def wait_out(b):
            for _ in range(_BW):
                pltpu.make_async_copy(
                    bufs[b].at[pl.ds(0, _PAD)],
                    o_hbm.at[0],
                    osems[b],
                ).wait()

        for b in range(_NBUF - 1):
            start_gather(b, b)

        @pl.loop(0, nwin // _NBUF)
        def _(p):
            for b in range(_NBUF):
                j = p * _NBUF + b
                wait_gather(b)
                start_out(j, b)
                gb = (b + _NBUF - 1) % _NBUF
                g = j + _NBUF - 1
                if b == 0:
                    @pl.when(p > 0)
                    def _():
                        wait_out(gb)
                    start_gather(g, gb)
                else:
                    wait_out(gb)

                    @pl.when(g < nwin)
                    def _():
                        start_gather(g, gb)

        wait_out((nwin - 1) % _NBUF)

    return gather_kernel


def _make_formatter(dtype, batch, cb, hist, dim, chunk):
    grid = (cb // _BB,)
    blk0 = chunk * (cb // _BB)

    def body(chunk_ref, out_ref):
        out_ref[...] = chunk_ref[...][:, :hist, :]

    def body_acc(acc_ref, chunk_ref, out_ref):
        del acc_ref
        out_ref[...] = chunk_ref[...][:, :hist, :]

    out_shape = jax.ShapeDtypeStruct((batch, hist, dim), dtype)
    chunk_spec = pl.BlockSpec((_BB, _PAD, dim), lambda i: (i, 0, 0))
    out_spec = pl.BlockSpec((_BB, hist, dim), lambda i: (blk0 + i, 0, 0))
    params = pltpu.CompilerParams(dimension_semantics=("parallel",))
    if chunk == 0:
        return pl.pallas_call(
            body,
            grid=grid,
            in_specs=[chunk_spec],
            out_specs=out_spec,
            out_shape=out_shape,
            compiler_params=params,
        )
    return pl.pallas_call(
        body_acc,
        grid=grid,
        in_specs=[pl.BlockSpec(memory_space=pl.ANY), chunk_spec],
        out_specs=out_spec,
        out_shape=out_shape,
        input_output_aliases={0: 0},
        compiler_params=params,
    )


def kernel(x, weight):
    batch, hist = x.shape
    dim = weight.shape[1]
    cb = batch // _NCHUNK
    bpw = cb // _NW
    xp = jnp.concatenate(
        [x.astype(jnp.int32), x[:, : _PAD - hist].astype(jnp.int32)], axis=1
    )
    idx = xp.reshape(_NCHUNK, _NW, bpw // _BW, _BW * _PAD)
    gather = _make_gather(weight.dtype, cb, hist, dim)
    chunks = [gather(weight, idx[c]) for c in range(_NCHUNK)]
    out = jnp.zeros((batch, hist, dim), weight.dtype)
    for c in range(_NCHUNK):
        out = lax.dynamic_update_slice(
            out, chunks[c][:, :hist, :], (c * cb, 0, 0)
        )
    return out


# speedup vs baseline: 1.8265x; 1.8265x over previous
import functools

import jax
import jax.numpy as jnp
from jax import lax
from jax.experimental import pallas as pl
from jax.experimental.pallas import tpu as pltpu
from jax.experimental.pallas import tpu_sc as plsc

_NW = 32
_BW = 2
_NBUF = 4


def _make_gather(dtype, batch, hist, dim):
    mesh = plsc.VectorSubcoreMesh(
        core_axis_name="core", subcore_axis_name="subcore"
    )
    bpw = batch // _NW
    nwin = bpw // _BW
    rows = _BW * hist

    @functools.partial(
        pl.kernel,
        mesh=mesh,
        out_type=jax.ShapeDtypeStruct((batch, hist, dim), dtype),
        scratch_types=[
            pltpu.VMEM((nwin, rows), jnp.int32),
        ]
        + [pltpu.VMEM((rows, dim), dtype) for _ in range(_NBUF)]
        + [pltpu.SemaphoreType.DMA for _ in range(2 * _NBUF)],
    )
    def gather_kernel(w_hbm, x_hbm, o_hbm, idx_v, *rest):
        bufs = rest[:_NBUF]
        gsems = rest[_NBUF:2 * _NBUF]
        osems = rest[2 * _NBUF:]

        wid = lax.axis_index("subcore") * 2 + lax.axis_index("core")
        base = wid * bpw
        pltpu.sync_copy(x_hbm.at[wid], idx_v)

        def start_gather(j, b):
            pltpu.async_copy(w_hbm.at[idx_v.at[j]], bufs[b], gsems[b])

        def wait_gather(b):
            pltpu.make_async_copy(
                w_hbm.at[idx_v.at[0]], bufs[b], gsems[b]
            ).wait()

        def start_out(j, b):
            b0 = base + j * _BW
            for k in range(_BW):
                pltpu.async_copy(
                    bufs[b].at[pl.ds(k * hist, hist)],
                    o_hbm.at[b0 + k],
                    osems[b],
                )

        def wait_out(b):
            for _ in range(_BW):
                pltpu.make_async_copy(
                    bufs[b].at[pl.ds(0, hist)], o_hbm.at[0], osems[b]
                ).wait()

        for b in range(_NBUF - 1):
            start_gather(b, b)

        @pl.loop(0, nwin // _NBUF)
        def _(p):
            for b in range(_NBUF):
                j = p * _NBUF + b
                wait_gather(b)
                start_out(j, b)
                gb = (b + _NBUF - 1) % _NBUF
                g = j + _NBUF - 1
                if b == 0:
                    @pl.when(p > 0)
                    def _():
                        wait_out(gb)
                    start_gather(g, gb)
                else:
                    wait_out(gb)

                    @pl.when(g < nwin)
                    def _():
                        start_gather(g, gb)

        wait_out((nwin - 1) % _NBUF)

    return gather_kernel


def kernel(x, weight):
    batch, hist = x.shape
    dim = weight.shape[1]
    bpw = batch // _NW
    idx = x.astype(jnp.int32).reshape(_NW, bpw // _BW, _BW * hist)
    return _make_gather(weight.dtype, batch, hist, dim)(weight, idx)
